# baseline (speedup 1.0000x reference)
import jax
import jax.numpy as jnp
from jax.experimental import pallas as pl
from jax.experimental.pallas import tpu as pltpu

OUT_FEATURES = 2048
SIZE = 28
LEVELS = 1024
LOW, HIGH = 0.0, 1.0
N = SIZE * SIZE


def _encode_kernel(xt_ref, pos_ref, lw_ref, out_ref):
    B = xt_ref.shape[1]
    idx = jnp.clip(
        jnp.round((xt_ref[...] - LOW) / (HIGH - LOW) * (LEVELS - 1)),
        0.0,
        LEVELS - 1.0,
    )

    s = lw_ref[0:1, :]
    e = lw_ref[LEVELS - 1:LEVELS, :]
    eq_start = jnp.where(lw_ref[...] == s, 1.0, 0.0)
    flip = jnp.sum(eq_start, axis=0, keepdims=True)
    pos = pos_ref[...]
    q = jnp.sum(pos, axis=0, keepdims=True)
    base = s * q
    r = e - s

    sub_iota = jax.lax.broadcasted_iota(jnp.int32, (B, 1), 0)

    def body(b, _):
        eb = jnp.where(sub_iota == b, 1.0, 0.0)
        ib = jnp.dot(
            idx,
            eb,
            precision=jax.lax.Precision.HIGHEST,
            preferred_element_type=jnp.float32,
        )
        contrib = jnp.where(ib >= flip, pos, 0.0)
        c = jnp.sum(contrib, axis=0, keepdims=True)
        ms = base + r * c
        row = jnp.where(ms > 0.0, 1.0, -1.0)
        out_ref[pl.ds(b, 1), :, :] = row[None]
        return 0

    jax.lax.fori_loop(0, B, body, 0)


def kernel(x, position_weight, level_weight):
    B = x.shape[0]
    flat_t = x.reshape(B, N).T
    out3 = pl.pallas_call(
        _encode_kernel,
        out_shape=jax.ShapeDtypeStruct((B, 1, OUT_FEATURES), jnp.float32),
    )(flat_t, position_weight, level_weight)
    return out3.reshape(B, OUT_FEATURES)


# speedup vs baseline: 1.3815x; 1.3815x over previous
import jax
import jax.numpy as jnp
from jax.experimental import pallas as pl
from jax.experimental.pallas import tpu as pltpu

OUT_FEATURES = 2048
SIZE = 28
LEVELS = 1024
LOW, HIGH = 0.0, 1.0
N = SIZE * SIZE


def _encode_kernel(xt_ref, pos_ref, lw_ref, out_ref):
    B = xt_ref.shape[1]
    idx = jnp.clip(
        jnp.round((xt_ref[...] - LOW) / (HIGH - LOW) * (LEVELS - 1)),
        0.0,
        LEVELS - 1.0,
    )

    s = lw_ref[0:1, :]
    e = lw_ref[LEVELS - 1:LEVELS, :]
    eq_start = jnp.where(lw_ref[...] == s, 1.0, 0.0)
    flip = jnp.sum(eq_start, axis=0, keepdims=True)
    pos = pos_ref[...]
    q = jnp.sum(pos, axis=0, keepdims=True)
    base = s * q
    r = e - s

    lane_iota = jax.lax.broadcasted_iota(jnp.int32, idx.shape, 1)
    n_rows = idx.shape[0]
    CH = 8

    def body(b, _):
        ib = jnp.sum(
            jnp.where(lane_iota == b, idx, 0.0), axis=1, keepdims=True
        )

        acc = jnp.zeros((CH, pos.shape[1]), jnp.float32)
        for k in range(n_rows // CH):
            ibk = ib[k * CH:(k + 1) * CH, :]
            posk = pos[k * CH:(k + 1) * CH, :]
            acc = acc + jnp.where(ibk >= flip, posk, 0.0)
        c = jnp.sum(acc, axis=0, keepdims=True)
        ms = base + r * c
        row = jnp.where(ms > 0.0, 1.0, -1.0)
        out_ref[pl.ds(b, 1), :, :] = row[None]
        return 0

    jax.lax.fori_loop(0, B, body, 0)


def kernel(x, position_weight, level_weight):
    B = x.shape[0]
    flat_t = x.reshape(B, N).T
    out3 = pl.pallas_call(
        _encode_kernel,
        out_shape=jax.ShapeDtypeStruct((B, 1, OUT_FEATURES), jnp.float32),
    )(flat_t, position_weight, level_weight)
    return out3.reshape(B, OUT_FEATURES)
